# baseline (speedup 1.0000x reference)
import functools

import jax
import jax.numpy as jnp
from jax import lax
from jax.experimental import pallas as pl
from jax.experimental.pallas import tpu as pltpu
from jax.experimental.pallas import tpu_sc as plsc

VOCAB = 1000
EMBED_DIM = 64
BATCH = 4096
SEQ = 200

NUM_CORES = 2
NUM_SUBCORES = 16
NW = NUM_CORES * NUM_SUBCORES
BCHUNK = 128
NBB = BATCH // BCHUNK
LANES = 16
NJ = EMBED_DIM // LANES

_mesh = plsc.VectorSubcoreMesh(core_axis_name="c", subcore_axis_name="s")


@functools.partial(
    pl.kernel,
    mesh=_mesh,
    out_type=jax.ShapeDtypeStruct((SEQ, EMBED_DIM, BATCH), jnp.float32),
    scratch_types=[
        pltpu.VMEM((VOCAB * EMBED_DIM,), jnp.float32),
        pltpu.VMEM((SEQ, BCHUNK), jnp.int32),
        pltpu.VMEM((2, 1, EMBED_DIM, BCHUNK), jnp.float32),
        pltpu.SemaphoreType.DMA,
        pltpu.SemaphoreType.DMA,
    ],
    compiler_params=pltpu.CompilerParams(use_tc_tiling_on_sc=True,
                                         needs_layout_passes=False),
)
def _sc_lookup(idx_hbm, table_hbm, out_hbm,
               table_v, idx_v, out_v, wsem_a, wsem_b):
    wid = lax.axis_index("s") * NUM_CORES + lax.axis_index("c")
    b0 = pl.multiple_of(wid * BCHUNK, BCHUNK)
    wsems = (wsem_a, wsem_b)
    iota = lax.iota(jnp.int32, LANES)
    rows = [iota + j * LANES for j in range(NJ)]
    zeros16 = jnp.zeros((LANES,), jnp.int32)

    pltpu.sync_copy(table_hbm, table_v)
    pltpu.sync_copy(idx_hbm.at[pl.ds(0, SEQ), pl.ds(b0, BCHUNK)], idx_v)

    def out_window(i):
        return out_hbm.at[pl.ds(i, 1), pl.ds(0, EMBED_DIM),
                          pl.ds(b0, BCHUNK)]

    def body(t, carry):
        for q in range(2):
            i = 2 * t + q

            @pl.when(t > 0)
            def _():
                pltpu.make_async_copy(out_v.at[q], out_window(0),
                                      wsems[q]).wait()

            @plsc.parallel_loop(0, BCHUNK, unroll=8)
            def fill(ii):
                bg = lax.shift_right_logical(ii, 4)
                r = lax.bitwise_and(ii, LANES - 1)
                bl = bg * LANES
                idx16 = idx_v[i, pl.ds(bl, LANES)]
                vbase = idx16 * EMBED_DIM
                colv = iota + bl
                rot = lax.bitwise_and(iota + r, LANES - 1)
                laddr = vbase + rot
                for j in range(NJ):
                    val = plsc.load_gather(table_v, [laddr + j * LANES])
                    plsc.store_scatter(out_v.at[q, 0],
                                       [rot + j * LANES, colv], val)

            pltpu.async_copy(out_v.at[q], out_window(i), wsems[q])
        return carry

    lax.fori_loop(0, SEQ // 2, body, 0)

    for q in range(2):
        pltpu.make_async_copy(out_v.at[q], out_window(0), wsems[q]).wait()


def kernel(word_sequences, table):
    idx_t = word_sequences.astype(jnp.int32).T
    table_flat = table.reshape(VOCAB * EMBED_DIM)
    out_t = _sc_lookup(idx_t, table_flat)
    return out_t.transpose(2, 0, 1)


# speedup vs baseline: 1.0457x; 1.0457x over previous
import functools

import jax
import jax.numpy as jnp
from jax import lax
from jax.experimental import pallas as pl
from jax.experimental.pallas import tpu as pltpu
from jax.experimental.pallas import tpu_sc as plsc

VOCAB = 1000
EMBED_DIM = 64
BATCH = 4096
SEQ = 200

NUM_CORES = 2
NUM_SUBCORES = 16
NW = NUM_CORES * NUM_SUBCORES
BCHUNK = 128
NBB = BATCH // BCHUNK
LANES = 16
NJ = EMBED_DIM // LANES

_mesh = plsc.VectorSubcoreMesh(core_axis_name="c", subcore_axis_name="s")


@functools.partial(
    pl.kernel,
    mesh=_mesh,
    out_type=jax.ShapeDtypeStruct((SEQ, EMBED_DIM, BATCH), jnp.float32),
    scratch_types=[
        pltpu.VMEM((VOCAB * EMBED_DIM,), jnp.float32),
        pltpu.VMEM((SEQ, BCHUNK), jnp.int32),
        pltpu.VMEM((2, 1, EMBED_DIM, BCHUNK), jnp.float32),
        pltpu.SemaphoreType.DMA,
        pltpu.SemaphoreType.DMA,
    ],
    compiler_params=pltpu.CompilerParams(use_tc_tiling_on_sc=True,
                                         needs_layout_passes=False),
)
def _sc_lookup(idx_hbm, table_hbm, out_hbm,
               table_v, idx_v, out_v, wsem_a, wsem_b):
    wid = lax.axis_index("s") * NUM_CORES + lax.axis_index("c")
    b0 = pl.multiple_of(wid * BCHUNK, BCHUNK)
    wsems = (wsem_a, wsem_b)
    iota = lax.iota(jnp.int32, LANES)
    rows = [iota + j * LANES for j in range(NJ)]
    zeros16 = jnp.zeros((LANES,), jnp.int32)

    pltpu.sync_copy(table_hbm, table_v)
    pltpu.sync_copy(idx_hbm.at[pl.ds(0, SEQ), pl.ds(b0, BCHUNK)], idx_v)

    def out_window(i):
        return out_hbm.at[pl.ds(i, 1), pl.ds(0, EMBED_DIM),
                          pl.ds(b0, BCHUNK)]

    def body(t, carry):
        for q in range(2):
            i = 2 * t + q

            @pl.when(t > 0)
            def _():
                pltpu.make_async_copy(out_v.at[q], out_window(0),
                                      wsems[q]).wait()

            @plsc.parallel_loop(0, BCHUNK, unroll=4)
            def fill(ii):
                bg = lax.shift_right_logical(ii, 4)
                r = lax.bitwise_and(ii, LANES - 1)
                bl = bg * LANES
                idx16 = idx_v[i, pl.ds(bl, LANES)]
                vbase = idx16 * EMBED_DIM
                colv = iota + bl
                rot = lax.bitwise_and(iota + r, LANES - 1)
                laddr = vbase + rot
                for j in range(NJ):
                    val = plsc.load_gather(table_v, [laddr + j * LANES])
                    plsc.store_scatter(out_v.at[q, 0],
                                       [rot + j * LANES, colv], val)

            pltpu.async_copy(out_v.at[q], out_window(i), wsems[q])
        return carry

    lax.fori_loop(0, SEQ // 2, body, 0)

    for q in range(2):
        pltpu.make_async_copy(out_v.at[q], out_window(0), wsems[q]).wait()


def kernel(word_sequences, table):
    idx_t = word_sequences.astype(jnp.int32).T
    table_flat = table.reshape(VOCAB * EMBED_DIM)
    out_t = _sc_lookup(idx_t, table_flat)
    return out_t.transpose(2, 0, 1)
